# baseline (speedup 1.0000x reference)
import functools

import jax
import jax.numpy as jnp
from jax import lax
from jax.experimental import pallas as pl
from jax.experimental.pallas import tpu as pltpu
from jax.experimental.pallas import tpu_sc as plsc

N = 10000
E = 160000
D = 256
H = 256
NCLS = 2

NC = 2
NS = 16
HALF = 128
CW = 16
NCNT = N + 16
W2PAD = 16
RB = 1000
GRID = N // RB

K1 = 128
NB1 = 80
EPT1 = E // NS
CNT_SPLIT = 40
K2 = 128
NB2 = 40
EPT2 = E // (NC * NS)
NCH = 5


def _tc_a_body(x_ref, w_ref, out_ref):
    out_ref[...] = lax.dot_general(
        x_ref[...], w_ref[...], (((1,), (1,)), ((), ())),
        preferred_element_type=jnp.float32).astype(jnp.bfloat16)


_tc_a = pl.pallas_call(
    _tc_a_body,
    grid=(NC * GRID,),
    in_specs=[
        pl.BlockSpec((RB, D), lambda j: (j % GRID, 0)),
        pl.BlockSpec((HALF, D), lambda j: (j // GRID, 0)),
    ],
    out_specs=pl.BlockSpec((RB, HALF), lambda j: (j, 0)),
    out_shape=jax.ShapeDtypeStruct((NC * N, HALF), jnp.bfloat16),
)


@functools.lru_cache(maxsize=None)
def _make_seg_sum(width, nb, k, shared_idx, src_stride, acc_rows, with_cnt,
                  dtype=jnp.float32):
    cb = nb // NCH
    rpt = acc_rows // NS
    crpt = NCNT // NS

    def body(table_ref, src_ref, dst_ref, *rest):
        if with_cnt:
            (out_ref, cnt_ref, src_a, src_b, dst_a, dst_b, rows0, rows1,
             ones, acc, acc_cnt, semi_a, semi_b, semg0, semg1) = rest
        else:
            (out_ref, src_a, src_b, dst_a, dst_b, rows0, rows1,
             acc, semi_a, semi_b, semg0, semg1) = rest
        c = lax.axis_index("c")
        s = lax.axis_index("s")

        vl = 16 if dtype == jnp.float32 else 32
        nlane = width // vl

        def _z(i, carry):
            r = i // nlane
            j = i % nlane
            rows0[r, pl.ds(j * vl, vl)] = jnp.zeros((vl,), dtype)
            return carry

        lax.fori_loop(0, k * nlane, _z, 0)
        nfull = rpt // k
        rem = rpt - nfull * k
        for q in range(nfull):
            pltpu.sync_copy(rows0, acc.at[pl.ds(s * rpt + q * k, k)])
        if rem:
            pltpu.sync_copy(rows0.at[pl.ds(0, rem)],
                            acc.at[pl.ds(s * rpt + nfull * k, rem)])

        if with_cnt:
            def _zc(i, carry):
                ones[i, pl.ds(0, 16)] = jnp.zeros((16,), jnp.float32)
                return carry

            lax.fori_loop(0, k, _zc, 0)
            cfull = crpt // k
            crem = crpt - cfull * k
            for q in range(cfull):
                pltpu.sync_copy(ones, acc_cnt.at[pl.ds(s * crpt + q * k, k)])
            if crem:
                pltpu.sync_copy(
                    ones.at[pl.ds(0, crem)],
                    acc_cnt.at[pl.ds(s * crpt + cfull * k, crem)])

            def _o(i, carry):
                ones[i, pl.ds(0, 16)] = jnp.ones((16,), jnp.float32)
                return carry

            lax.fori_loop(0, k, _o, 0)

        def _src_chunk(ch):
            if shared_idx:
                return src_ref.at[s, pl.ds(ch * cb, cb)]
            return src_ref.at[c, s, pl.ds(ch * cb, cb)]

        def _dst_chunk(ch):
            if shared_idx:
                return dst_ref.at[s, pl.ds(ch * cb, cb)]
            return dst_ref.at[c, s, pl.ds(ch * cb, cb)]

        pltpu.sync_copy(_src_chunk(0), src_a)
        pltpu.sync_copy(_dst_chunk(0), dst_a)
        plsc.subcore_barrier()

        bufs = [(src_a, dst_a, semi_a), (src_b, dst_b, semi_b)]
        for ch in range(NCH):
            sbuf, dbuf, semi = bufs[ch % 2]
            sbuf_n, dbuf_n, semi_n = bufs[(ch + 1) % 2]
            if ch > 0:
                pltpu.make_async_copy(_src_chunk(ch), sbuf, semi).wait()
                pltpu.make_async_copy(_dst_chunk(ch), dbuf, semi).wait()
            if ch + 1 < NCH:
                pltpu.async_copy(_src_chunk(ch + 1), sbuf_n, semi_n)
                pltpu.async_copy(_dst_chunk(ch + 1), dbuf_n, semi_n)

            if shared_idx:
                coff = c * src_stride

                def _ofs(i, carry, sbuf=sbuf):
                    r = i // (k // 16)
                    j = i % (k // 16)
                    sbuf[r, pl.ds(j * 16, 16)] = (
                        sbuf[r, pl.ds(j * 16, 16)] + coff)
                    return carry

                lax.fori_loop(0, cb * (k // 16), _ofs, 0)

            pltpu.async_copy(table_ref.at[sbuf.at[0]], rows0, semg0)

            def _blk(b, carry, sbuf=sbuf, dbuf=dbuf, ch=ch):
                @pl.when(jnp.logical_and(b + 1 < cb, (b + 1) % 2 == 0))
                def _():
                    pltpu.async_copy(table_ref.at[sbuf.at[b + 1]], rows0,
                                     semg0)

                @pl.when(jnp.logical_and(b + 1 < cb, (b + 1) % 2 == 1))
                def _():
                    pltpu.async_copy(table_ref.at[sbuf.at[b + 1]], rows1,
                                     semg1)

                if with_cnt:
                    inlow = (ch * cb + b) < CNT_SPLIT
                    mine = jnp.where(c == 0, inlow, jnp.logical_not(inlow))

                    @pl.when(mine)
                    def _():
                        pltpu.sync_copy(ones, acc_cnt.at[dbuf.at[b]],
                                        add=True)

                @pl.when(b % 2 == 0)
                def _():
                    pltpu.make_async_copy(
                        table_ref.at[sbuf.at[b]], rows0, semg0).wait()
                    pltpu.sync_copy(rows0, acc.at[dbuf.at[b]], add=True)

                @pl.when(b % 2 == 1)
                def _():
                    pltpu.make_async_copy(
                        table_ref.at[sbuf.at[b]], rows1, semg1).wait()
                    pltpu.sync_copy(rows1, acc.at[dbuf.at[b]], add=True)

                return carry

            lax.fori_loop(0, cb, _blk, 0)

        plsc.subcore_barrier()
        pltpu.sync_copy(acc.at[pl.ds(s * rpt, rpt)],
                        out_ref.at[c, pl.ds(s * rpt, rpt)])
        if with_cnt:
            pltpu.sync_copy(acc_cnt.at[pl.ds(s * crpt, crpt)],
                            cnt_ref.at[c, pl.ds(s * crpt, crpt)])

    out_type = [jax.ShapeDtypeStruct((NC, acc_rows, width), dtype)]
    scratch = [
        pltpu.VMEM((cb, k), jnp.int32),
        pltpu.VMEM((cb, k), jnp.int32),
        pltpu.VMEM((cb, k), jnp.int32),
        pltpu.VMEM((cb, k), jnp.int32),
        pltpu.VMEM((k, width), dtype),
        pltpu.VMEM((k, width), dtype),
    ]
    if with_cnt:
        out_type.append(jax.ShapeDtypeStruct((NC, NCNT, CW), jnp.float32))
        scratch.append(pltpu.VMEM((k, CW), jnp.float32))
    scratch.append(pltpu.VMEM_SHARED((acc_rows, width), dtype))
    if with_cnt:
        scratch.append(pltpu.VMEM_SHARED((NCNT, CW), jnp.float32))
    scratch += [pltpu.SemaphoreType.DMA] * 4

    return pl.kernel(
        body,
        out_type=out_type,
        mesh=plsc.VectorSubcoreMesh(core_axis_name="c", subcore_axis_name="s"),
        scratch_types=scratch,
        compiler_params=pltpu.CompilerParams(use_tc_tiling_on_sc=False),
    )


def _tc_b_body(seg_ref, cnt_ref, x_ref, w1r_ref, b1_ref, w2l_ref, w2r_ref,
               b2_ref, qaug_ref, r2c_ref):
    sums = jnp.concatenate(
        [seg_ref[0], seg_ref[1]], axis=1).astype(jnp.float32)
    denom = jnp.maximum(cnt_ref[0, :, :1] + cnt_ref[1, :, :1], 1.0)
    r = lax.dot_general(x_ref[...], w1r_ref[...], (((1,), (1,)), ((), ())),
                        preferred_element_type=jnp.float32) + b1_ref[...]
    h = jnp.maximum(sums / denom + r, 0.0)
    q = lax.dot_general(h, w2l_ref[...], (((1,), (1,)), ((), ())),
                        preferred_element_type=jnp.float32)
    r2 = lax.dot_general(h, w2r_ref[...], (((1,), (1,)), ((), ())),
                         preferred_element_type=jnp.float32) + b2_ref[...]
    qaug_ref[...] = q
    r2c_ref[...] = jnp.concatenate(
        [r2[:, :NCLS], denom, jnp.zeros((RB, W2PAD - NCLS - 1), jnp.float32)],
        axis=1)


_tc_b = pl.pallas_call(
    _tc_b_body,
    grid=(GRID,),
    in_specs=[
        pl.BlockSpec((NC, RB, HALF), lambda i: (0, i, 0)),
        pl.BlockSpec((NC, RB, CW), lambda i: (0, i, 0)),
        pl.BlockSpec((RB, D), lambda i: (i, 0)),
        pl.BlockSpec((H, D), lambda i: (0, 0)),
        pl.BlockSpec((1, H), lambda i: (0, 0)),
        pl.BlockSpec((W2PAD, H), lambda i: (0, 0)),
        pl.BlockSpec((W2PAD, H), lambda i: (0, 0)),
        pl.BlockSpec((1, W2PAD), lambda i: (0, 0)),
    ],
    out_specs=[
        pl.BlockSpec((RB, W2PAD), lambda i: (i, 0)),
        pl.BlockSpec((RB, W2PAD), lambda i: (i, 0)),
    ],
    out_shape=[
        jax.ShapeDtypeStruct((N, W2PAD), jnp.float32),
        jax.ShapeDtypeStruct((N, W2PAD), jnp.float32),
    ],
)


def _tc_c_body(seg2_ref, r2c_ref, out_ref):
    s2 = seg2_ref[0] + seg2_ref[1]
    r2c = r2c_ref[...]
    out_ref[...] = s2[:, :NCLS] / r2c[:, NCLS:NCLS + 1] + r2c[:, :NCLS]


RBC = 2000

_tc_c = pl.pallas_call(
    _tc_c_body,
    grid=(N // RBC,),
    in_specs=[
        pl.BlockSpec((NC, RBC, W2PAD), lambda i: (0, i, 0)),
        pl.BlockSpec((RBC, W2PAD), lambda i: (i, 0)),
    ],
    out_specs=pl.BlockSpec((RBC, NCLS), lambda i: (i, 0)),
    out_shape=jax.ShapeDtypeStruct((N, NCLS), jnp.float32),
)


def kernel(x, edge_index, W1l, b1, W1r, W2l, b2, W2r):
    src = edge_index[0].astype(jnp.int32)
    dst = edge_index[1].astype(jnp.int32)

    table1 = _tc_a(x, W1l)

    padr = ((0, 0), (0, NB1 * K1 - EPT1))
    src3 = jnp.pad(src.reshape(NS, EPT1), padr).reshape(NS, NB1, K1)
    dst3 = jnp.pad(dst.reshape(NS, EPT1), padr,
                   constant_values=N).reshape(NS, NB1, K1)
    seg1, cnt1 = _make_seg_sum(HALF, NB1, K1, True, N, NCNT, True,
                               jnp.bfloat16)(table1, src3, dst3)

    w2lp = jnp.zeros((W2PAD, H), jnp.float32).at[:NCLS].set(W2l)
    w2rp = jnp.zeros((W2PAD, H), jnp.float32).at[:NCLS].set(W2r)
    b2p = jnp.zeros((1, W2PAD), jnp.float32).at[0, :NCLS].set(b2)
    qaug, r2c = _tc_b(seg1, cnt1, x, W1r, b1.reshape(1, H), w2lp, w2rp, b2p)

    padc = ((0, 0), (0, NB2 * K2 - EPT2))
    src2 = jnp.pad(src.reshape(NC * NS, EPT2), padc).reshape(NC, NS, NB2, K2)
    dst2 = jnp.pad(dst.reshape(NC * NS, EPT2), padc,
                   constant_values=N).reshape(NC, NS, NB2, K2)
    (seg2,) = _make_seg_sum(W2PAD, NB2, K2, False, 0, NCNT, False)(
        qaug, src2, dst2)

    return _tc_c(seg2, r2c)


# speedup vs baseline: 1.2683x; 1.2683x over previous
import functools

import jax
import jax.numpy as jnp
from jax import lax
from jax.experimental import pallas as pl
from jax.experimental.pallas import tpu as pltpu
from jax.experimental.pallas import tpu_sc as plsc

N = 10000
E = 160000
D = 256
H = 256
NCLS = 2

NC = 2
NS = 16
HALF = 128
CW = 16
NCNT = N + 16
W2PAD = 16
RB = 1000
GRID = N // RB

K1 = 128
NB1 = 80
EPT1 = E // NS
CNT_SPLIT = 40
K2 = 128
NB2 = 40
EPT2 = E // (NC * NS)
NCH = 5


def _tc_a_body(x_ref, w_ref, out_ref):
    out_ref[...] = lax.dot_general(
        x_ref[...], w_ref[...], (((1,), (1,)), ((), ())),
        preferred_element_type=jnp.float32).astype(jnp.bfloat16)


_tc_a = pl.pallas_call(
    _tc_a_body,
    grid=(NC * GRID,),
    in_specs=[
        pl.BlockSpec((RB, D), lambda j: (j % GRID, 0)),
        pl.BlockSpec((HALF, D), lambda j: (j // GRID, 0)),
    ],
    out_specs=pl.BlockSpec((RB, HALF), lambda j: (j, 0)),
    out_shape=jax.ShapeDtypeStruct((NC * N, HALF), jnp.bfloat16),
)


@functools.lru_cache(maxsize=None)
def _make_seg_sum(width, nb, k, shared_idx, src_stride, acc_rows, with_cnt,
                  dtype=jnp.float32):
    cb = nb // NCH
    rpt = acc_rows // NS
    crpt = NCNT // NS

    def body(table_ref, src_ref, dst_ref, *rest):
        if with_cnt:
            (out_ref, cnt_ref, src_a, src_b, dst_a, dst_b, rows0, rows1,
             ones, acc, acc_cnt, semi_a, semi_b, semg0, semg1) = rest
        else:
            (out_ref, src_a, src_b, dst_a, dst_b, rows0, rows1,
             acc, semi_a, semi_b, semg0, semg1) = rest
        c = lax.axis_index("c")
        s = lax.axis_index("s")

        vl = 16 if dtype == jnp.float32 else 32
        nlane = width // vl

        def _z(i, carry):
            r = i // nlane
            j = i % nlane
            rows0[r, pl.ds(j * vl, vl)] = jnp.zeros((vl,), dtype)
            return carry

        lax.fori_loop(0, k * nlane, _z, 0)
        nfull = rpt // k
        rem = rpt - nfull * k
        for q in range(nfull):
            pltpu.sync_copy(rows0, acc.at[pl.ds(s * rpt + q * k, k)])
        if rem:
            pltpu.sync_copy(rows0.at[pl.ds(0, rem)],
                            acc.at[pl.ds(s * rpt + nfull * k, rem)])

        if with_cnt:
            def _zc(i, carry):
                ones[i, pl.ds(0, 16)] = jnp.zeros((16,), jnp.float32)
                return carry

            lax.fori_loop(0, k, _zc, 0)
            cfull = crpt // k
            crem = crpt - cfull * k
            for q in range(cfull):
                pltpu.sync_copy(ones, acc_cnt.at[pl.ds(s * crpt + q * k, k)])
            if crem:
                pltpu.sync_copy(
                    ones.at[pl.ds(0, crem)],
                    acc_cnt.at[pl.ds(s * crpt + cfull * k, crem)])

            def _o(i, carry):
                ones[i, pl.ds(0, 16)] = jnp.ones((16,), jnp.float32)
                return carry

            lax.fori_loop(0, k, _o, 0)

        def _src_chunk(ch):
            if shared_idx:
                return src_ref.at[s, pl.ds(ch * cb, cb)]
            return src_ref.at[c, s, pl.ds(ch * cb, cb)]

        def _dst_chunk(ch):
            if shared_idx:
                return dst_ref.at[s, pl.ds(ch * cb, cb)]
            return dst_ref.at[c, s, pl.ds(ch * cb, cb)]

        pltpu.sync_copy(_src_chunk(0), src_a)
        pltpu.sync_copy(_dst_chunk(0), dst_a)
        plsc.subcore_barrier()

        bufs = [(src_a, dst_a, semi_a), (src_b, dst_b, semi_b)]
        for ch in range(NCH):
            sbuf, dbuf, semi = bufs[ch % 2]
            sbuf_n, dbuf_n, semi_n = bufs[(ch + 1) % 2]
            if ch > 0:
                pltpu.make_async_copy(_src_chunk(ch), sbuf, semi).wait()
                pltpu.make_async_copy(_dst_chunk(ch), dbuf, semi).wait()
            if ch + 1 < NCH:
                pltpu.async_copy(_src_chunk(ch + 1), sbuf_n, semi_n)
                pltpu.async_copy(_dst_chunk(ch + 1), dbuf_n, semi_n)

            if shared_idx:
                coff = c * src_stride

                def _ofs(i, carry, sbuf=sbuf):
                    r = i // (k // 16)
                    j = i % (k // 16)
                    sbuf[r, pl.ds(j * 16, 16)] = (
                        sbuf[r, pl.ds(j * 16, 16)] + coff)
                    return carry

                lax.fori_loop(0, cb * (k // 16), _ofs, 0)

            pltpu.async_copy(table_ref.at[sbuf.at[0]], rows0, semg0)

            def _blk(b, carry, sbuf=sbuf, dbuf=dbuf, ch=ch):
                @pl.when(jnp.logical_and(b + 1 < cb, (b + 1) % 2 == 0))
                def _():
                    pltpu.async_copy(table_ref.at[sbuf.at[b + 1]], rows0,
                                     semg0)

                @pl.when(jnp.logical_and(b + 1 < cb, (b + 1) % 2 == 1))
                def _():
                    pltpu.async_copy(table_ref.at[sbuf.at[b + 1]], rows1,
                                     semg1)

                if with_cnt:
                    inlow = (ch * cb + b) < CNT_SPLIT
                    mine = jnp.where(c == 0, inlow, jnp.logical_not(inlow))

                    @pl.when(mine)
                    def _():
                        pltpu.sync_copy(ones, acc_cnt.at[dbuf.at[b]],
                                        add=True)

                @pl.when(b % 2 == 0)
                def _():
                    pltpu.make_async_copy(
                        table_ref.at[sbuf.at[b]], rows0, semg0).wait()
                    pltpu.sync_copy(rows0, acc.at[dbuf.at[b]], add=True)

                @pl.when(b % 2 == 1)
                def _():
                    pltpu.make_async_copy(
                        table_ref.at[sbuf.at[b]], rows1, semg1).wait()
                    pltpu.sync_copy(rows1, acc.at[dbuf.at[b]], add=True)

                return carry

            lax.fori_loop(0, cb, _blk, 0)

        plsc.subcore_barrier()
        pltpu.sync_copy(acc.at[pl.ds(s * rpt, rpt)],
                        out_ref.at[c, pl.ds(s * rpt, rpt)])
        if with_cnt:
            pltpu.sync_copy(acc_cnt.at[pl.ds(s * crpt, crpt)],
                            cnt_ref.at[c, pl.ds(s * crpt, crpt)])

    out_type = [jax.ShapeDtypeStruct((NC, acc_rows, width), dtype)]
    scratch = [
        pltpu.VMEM((cb, k), jnp.int32),
        pltpu.VMEM((cb, k), jnp.int32),
        pltpu.VMEM((cb, k), jnp.int32),
        pltpu.VMEM((cb, k), jnp.int32),
        pltpu.VMEM((k, width), dtype),
        pltpu.VMEM((k, width), dtype),
    ]
    if with_cnt:
        out_type.append(jax.ShapeDtypeStruct((NC, NCNT, CW), jnp.float32))
        scratch.append(pltpu.VMEM((k, CW), jnp.float32))
    scratch.append(pltpu.VMEM_SHARED((acc_rows, width), dtype))
    if with_cnt:
        scratch.append(pltpu.VMEM_SHARED((NCNT, CW), jnp.float32))
    scratch += [pltpu.SemaphoreType.DMA] * 4

    return pl.kernel(
        body,
        out_type=out_type,
        mesh=plsc.VectorSubcoreMesh(core_axis_name="c", subcore_axis_name="s"),
        scratch_types=scratch,
        compiler_params=pltpu.CompilerParams(use_tc_tiling_on_sc=False),
    )


def _tc_b_body(seg_ref, cnt_ref, x_ref, w1r_ref, b1_ref, w2l_ref, w2r_ref,
               b2_ref, qaug_ref, r2c_ref):
    sums = jnp.concatenate(
        [seg_ref[0], seg_ref[1]], axis=1).astype(jnp.float32)
    denom = jnp.maximum(cnt_ref[0, :, :1] + cnt_ref[1, :, :1], 1.0)
    r = lax.dot_general(x_ref[...], w1r_ref[...], (((1,), (1,)), ((), ())),
                        preferred_element_type=jnp.float32) + b1_ref[...]
    h = jnp.maximum(sums / denom + r, 0.0)
    q = lax.dot_general(h, w2l_ref[...], (((1,), (1,)), ((), ())),
                        preferred_element_type=jnp.float32)
    r2 = lax.dot_general(h, w2r_ref[...], (((1,), (1,)), ((), ())),
                         preferred_element_type=jnp.float32) + b2_ref[...]
    qaug_ref[...] = q
    r2c_ref[...] = jnp.concatenate(
        [r2[:, :NCLS], denom, jnp.zeros((RB, W2PAD - NCLS - 1), jnp.float32)],
        axis=1)


_tc_b = pl.pallas_call(
    _tc_b_body,
    grid=(GRID,),
    in_specs=[
        pl.BlockSpec((NC, RB, HALF), lambda i: (0, i, 0)),
        pl.BlockSpec((NC, RB, CW), lambda i: (0, i, 0)),
        pl.BlockSpec((RB, D), lambda i: (i, 0)),
        pl.BlockSpec((H, D), lambda i: (0, 0)),
        pl.BlockSpec((1, H), lambda i: (0, 0)),
        pl.BlockSpec((W2PAD, H), lambda i: (0, 0)),
        pl.BlockSpec((W2PAD, H), lambda i: (0, 0)),
        pl.BlockSpec((1, W2PAD), lambda i: (0, 0)),
    ],
    out_specs=[
        pl.BlockSpec((RB, W2PAD), lambda i: (i, 0)),
        pl.BlockSpec((RB, W2PAD), lambda i: (i, 0)),
    ],
    out_shape=[
        jax.ShapeDtypeStruct((N, W2PAD), jnp.float32),
        jax.ShapeDtypeStruct((N, W2PAD), jnp.float32),
    ],
)


def _tc_c_body(seg2_ref, r2c_ref, out_ref):
    s2 = seg2_ref[0] + seg2_ref[1]
    r2c = r2c_ref[...]
    out_ref[...] = s2[:, :NCLS] / r2c[:, NCLS:NCLS + 1] + r2c[:, :NCLS]


RBC = 2000

_tc_c = pl.pallas_call(
    _tc_c_body,
    grid=(N // RBC,),
    in_specs=[
        pl.BlockSpec((NC, RBC, W2PAD), lambda i: (0, i, 0)),
        pl.BlockSpec((RBC, W2PAD), lambda i: (i, 0)),
    ],
    out_specs=pl.BlockSpec((RBC, NCLS), lambda i: (i, 0)),
    out_shape=jax.ShapeDtypeStruct((N, NCLS), jnp.float32),
)


def kernel(x, edge_index, W1l, b1, W1r, W2l, b2, W2r):
    src = edge_index[0].astype(jnp.int32)
    dst = edge_index[1].astype(jnp.int32)

    table1 = _tc_a(x, W1l)

    npad1 = NB1 * K1 - EPT1
    sgarb = jnp.broadcast_to(jnp.arange(npad1, dtype=jnp.int32) * 37 % N,
                             (NS, npad1))
    dgarb = jnp.broadcast_to(
        N + (jnp.arange(npad1, dtype=jnp.int32) % 16), (NS, npad1))
    src3 = jnp.concatenate([src.reshape(NS, EPT1), sgarb],
                           axis=1).reshape(NS, NB1, K1)
    dst3 = jnp.concatenate([dst.reshape(NS, EPT1), dgarb],
                           axis=1).reshape(NS, NB1, K1)
    seg1, cnt1 = _make_seg_sum(HALF, NB1, K1, True, N, NCNT, True,
                               jnp.bfloat16)(table1, src3, dst3)

    w2lp = jnp.zeros((W2PAD, H), jnp.float32).at[:NCLS].set(W2l)
    w2rp = jnp.zeros((W2PAD, H), jnp.float32).at[:NCLS].set(W2r)
    b2p = jnp.zeros((1, W2PAD), jnp.float32).at[0, :NCLS].set(b2)
    qaug, r2c = _tc_b(seg1, cnt1, x, W1r, b1.reshape(1, H), w2lp, w2rp, b2p)

    padc = ((0, 0), (0, NB2 * K2 - EPT2))
    src2 = jnp.pad(src.reshape(NC * NS, EPT2), padc).reshape(NC, NS, NB2, K2)
    dst2 = jnp.pad(dst.reshape(NC * NS, EPT2), padc,
                   constant_values=N).reshape(NC, NS, NB2, K2)
    (seg2,) = _make_seg_sum(W2PAD, NB2, K2, False, 0, NCNT, False)(
        qaug, src2, dst2)

    return _tc_c(seg2, r2c)
